# Initial kernel scaffold; baseline (speedup 1.0000x reference)
import jax
import jax.numpy as jnp
from jax.experimental import pallas as pl


def kernel(user_emb, item_emb, edge_index1, edge_vals1, edge_index2, edge_vals2, edge_index3, edge_vals3):
    raise NotImplementedError("write your pallas kernel here")


# speedup vs baseline: 8.7264x; 8.7264x over previous
import functools

import jax
import jax.numpy as jnp
from jax import lax
from jax.experimental import pallas as pl
from jax.experimental.pallas import tpu as pltpu
from jax.experimental.pallas import tpu_sc as plsc

N_USERS = 50000
N = 100000
D = 32
H = 16
E = 1600000
EP = 1638400
NBLK = EP // 128
TILES = 16
EDGES_PER_TILE = EP // TILES
CHUNK = 1024
BLK_PER_CHUNK = CHUNK // 128
CHUNKS_PER_TILE = EDGES_PER_TILE // CHUNK
ROWS_BIG = 6256
ROWS_SMALL = 6248


def _layer_body(table, colsb, rows3, valsf, zeros, out,
                gidx_v, sidx_v, vals_v, rows_buf, acc, gsem, ssem):
    c = lax.axis_index("c")
    s = lax.axis_index("s")

    def _row_range_copy(fn_big, fn_small):
        @pl.when(s < 4)
        def _():
            lo = pl.multiple_of(s * ROWS_BIG, 8)
            fn_big(lo)

        @pl.when(s >= 4)
        def _():
            lo = pl.multiple_of(4 * ROWS_BIG + (s - 4) * ROWS_SMALL, 8)
            fn_small(lo)

    _row_range_copy(
        lambda lo: pltpu.sync_copy(zeros.at[pl.ds(0, ROWS_BIG)],
                                   acc.at[pl.ds(lo, ROWS_BIG)]),
        lambda lo: pltpu.sync_copy(zeros.at[pl.ds(0, ROWS_SMALL)],
                                   acc.at[pl.ds(lo, ROWS_SMALL)]))
    plsc.subcore_barrier()

    def chunk_body(g, _):
        blk = s * (EDGES_PER_TILE // 128) + g * BLK_PER_CHUNK
        pltpu.sync_copy(colsb.at[pl.ds(c * NBLK + blk, BLK_PER_CHUNK)], gidx_v)
        pltpu.sync_copy(rows3.at[pl.ds(blk, BLK_PER_CHUNK)], sidx_v)
        pltpu.sync_copy(valsf.at[pl.ds(blk * 128, CHUNK)], vals_v)

        descs = []
        for j in range(BLK_PER_CHUNK):
            descs.append(pltpu.async_copy(
                table.at[gidx_v.at[j]],
                rows_buf.at[pl.ds(j * 128, 128)], gsem))
        for d in descs:
            d.wait()

        def scale_body(k, _):
            v16 = vals_v[pl.ds(k * 16, 16)]
            base = k * 16
            for i in range(16):
                e = base + i
                scale = lax.gather(
                    v16, jnp.full((16, 1), i, jnp.int32),
                    lax.GatherDimensionNumbers(offset_dims=(),
                                               collapsed_slice_dims=(0,),
                                               start_index_map=(0,)),
                    slice_sizes=(1,),
                    mode=lax.GatherScatterMode.PROMISE_IN_BOUNDS)
                rows_buf[e, :] = rows_buf[e, :] * scale
            return 0
        lax.fori_loop(0, CHUNK // 16, scale_body, 0)

        descs = []
        for j in range(BLK_PER_CHUNK):
            descs.append(pltpu.async_copy(
                rows_buf.at[pl.ds(j * 128, 128)],
                acc.at[sidx_v.at[j]], ssem, add=True))
        for d in descs:
            d.wait()
        return 0

    lax.fori_loop(0, CHUNKS_PER_TILE, chunk_body, 0)
    plsc.subcore_barrier()

    _row_range_copy(
        lambda lo: pltpu.sync_copy(
            acc.at[pl.ds(lo, ROWS_BIG)],
            out.at[pl.ds(pl.multiple_of(c * N + lo, 8), ROWS_BIG)]),
        lambda lo: pltpu.sync_copy(
            acc.at[pl.ds(lo, ROWS_SMALL)],
            out.at[pl.ds(pl.multiple_of(c * N + lo, 8), ROWS_SMALL)]))


@jax.jit
def _propagate_layer(table, colsb, rows3, valsf, zeros):
    mesh = plsc.VectorSubcoreMesh(core_axis_name="c", subcore_axis_name="s")
    return pl.kernel(
        _layer_body,
        out_type=jax.ShapeDtypeStruct((2 * N, H), jnp.float32),
        mesh=mesh,
        scratch_types=[
            pltpu.VMEM((BLK_PER_CHUNK, 128), jnp.int32),
            pltpu.VMEM((BLK_PER_CHUNK, 128), jnp.int32),
            pltpu.VMEM((CHUNK,), jnp.float32),
            pltpu.VMEM((CHUNK, H), jnp.float32),
            pltpu.VMEM_SHARED((N, H), jnp.float32),
            pltpu.SemaphoreType.DMA,
            pltpu.SemaphoreType.DMA,
        ],
        compiler_params=pltpu.CompilerParams(use_tc_tiling_on_sc=False),
    )(table, colsb, rows3, valsf, zeros)


def _prep_edges(edge_index, edge_vals):
    pad = EP - E
    rows = jnp.concatenate([edge_index[0], jnp.zeros((pad,), jnp.int32)])
    cols = jnp.concatenate([edge_index[1], jnp.zeros((pad,), jnp.int32)])
    vals = jnp.concatenate([edge_vals, jnp.zeros((pad,), jnp.float32)])
    colsb = jnp.concatenate([cols, cols + N]).reshape(2 * NBLK, 128)
    rows3 = rows.reshape(NBLK, 128)
    return colsb, rows3, vals


def _split(x):
    return x.reshape(N, 2, H).transpose(1, 0, 2).reshape(2 * N, H)


def _unsplit(y):
    return y.reshape(2, N, H).transpose(1, 0, 2).reshape(N, D)


def kernel(user_emb, item_emb, edge_index1, edge_vals1,
           edge_index2, edge_vals2, edge_index3, edge_vals3):
    zeros = jnp.zeros((ROWS_BIG, H), jnp.float32)
    ego = _split(jnp.concatenate([user_emb, item_emb], axis=0))

    stage_outs = []
    for ei, ev in ((edge_index3, edge_vals3),
                   (edge_index2, edge_vals2),
                   (edge_index1, edge_vals1)):
        colsb, rows3, vals = _prep_edges(ei, ev)
        a1 = _propagate_layer(ego, colsb, rows3, vals, zeros)
        a2 = _propagate_layer(a1, colsb, rows3, vals, zeros)
        ego = (ego + a1 + a2) * (1.0 / 3.0)
        stage_outs.append(ego)

    total = _unsplit(stage_outs[0] + stage_outs[1] + stage_outs[2])
    return total[:N_USERS], total[N_USERS:]


# speedup vs baseline: 10.6738x; 1.2232x over previous
import functools

import jax
import jax.numpy as jnp
from jax import lax
from jax.experimental import pallas as pl
from jax.experimental.pallas import tpu as pltpu
from jax.experimental.pallas import tpu_sc as plsc

N_USERS = 50000
N = 100000
D = 32
H = 16
E = 1600000
EP = 1638400
NBLK = EP // 128
TILES = 16
EDGES_PER_TILE = EP // TILES
CHUNK = 512
BLK_PER_CHUNK = CHUNK // 128
CHUNKS_PER_TILE = EDGES_PER_TILE // CHUNK
GATHER_BYTES = CHUNK * H * 4
SCATTER_BYTES = CHUNK * H * 4
IDX_BYTES = 3 * CHUNK * 4
ROWS_BIG = 6256
ROWS_SMALL = 6248


def _layer_body(table, colsb, rows3, valsf, zeros, out,
                gidx_v, sidx_v, vals_v, rows_buf, acc, gsem, ssem, isem):
    c = lax.axis_index("c")
    s = lax.axis_index("s")

    def _row_range_copy(fn_big, fn_small):
        @pl.when(s < 4)
        def _():
            lo = pl.multiple_of(s * ROWS_BIG, 8)
            fn_big(lo)

        @pl.when(s >= 4)
        def _():
            lo = pl.multiple_of(4 * ROWS_BIG + (s - 4) * ROWS_SMALL, 8)
            fn_small(lo)

    _row_range_copy(
        lambda lo: pltpu.sync_copy(zeros.at[pl.ds(0, ROWS_BIG)],
                                   acc.at[pl.ds(lo, ROWS_BIG)]),
        lambda lo: pltpu.sync_copy(zeros.at[pl.ds(0, ROWS_SMALL)],
                                   acc.at[pl.ds(lo, ROWS_SMALL)]))
    plsc.subcore_barrier()

    NC = CHUNKS_PER_TILE

    def drain_gather():
        pltpu.make_async_copy(table.at[pl.ds(0, CHUNK)],
                              rows_buf.at[pl.ds(0, CHUNK)], gsem).wait()

    def drain_scatter():
        pltpu.make_async_copy(table.at[pl.ds(0, CHUNK)],
                              rows_buf.at[pl.ds(0, CHUNK)], ssem).wait()

    def drain_idx():
        pltpu.make_async_copy(valsf.at[pl.ds(0, 3 * CHUNK)], vals_v,
                              isem).wait()

    def stage_idx(g):
        st = lax.rem(g, 3)
        blk = s * (EDGES_PER_TILE // 128) + g * BLK_PER_CHUNK
        pltpu.async_copy(colsb.at[pl.ds(c * NBLK + blk, BLK_PER_CHUNK)],
                         gidx_v.at[pl.ds(st * BLK_PER_CHUNK, BLK_PER_CHUNK)],
                         isem)
        pltpu.async_copy(rows3.at[pl.ds(blk, BLK_PER_CHUNK)],
                         sidx_v.at[pl.ds(st * BLK_PER_CHUNK, BLK_PER_CHUNK)],
                         isem)
        pltpu.async_copy(valsf.at[pl.ds(blk * 128, CHUNK)],
                         vals_v.at[pl.ds(st * CHUNK, CHUNK)], isem)

    def issue_gathers(g):
        st = lax.rem(g, 3)
        b = lax.rem(g, 2)
        for j in range(BLK_PER_CHUNK):
            pltpu.async_copy(
                table.at[gidx_v.at[st * BLK_PER_CHUNK + j]],
                rows_buf.at[pl.ds(b * CHUNK + j * 128, 128)], gsem)

    def issue_scatters(g):
        st = lax.rem(g, 3)
        b = lax.rem(g, 2)
        for j in range(BLK_PER_CHUNK):
            pltpu.async_copy(
                rows_buf.at[pl.ds(b * CHUNK + j * 128, 128)],
                acc.at[sidx_v.at[st * BLK_PER_CHUNK + j]], ssem, add=True)

    stage_idx(jnp.int32(0))
    drain_idx()
    issue_gathers(jnp.int32(0))
    stage_idx(jnp.int32(1))

    def chunk_body(g, _):
        st = lax.rem(g, 3)
        b = lax.rem(g, 2)
        drain_gather()

        def scale_body(k, _):
            v16 = vals_v[pl.ds(st * CHUNK + k * 16, 16)]
            base = b * CHUNK + k * 16
            for i in range(16):
                e = base + i
                scale = lax.gather(
                    v16, jnp.full((16, 1), i, jnp.int32),
                    lax.GatherDimensionNumbers(offset_dims=(),
                                               collapsed_slice_dims=(0,),
                                               start_index_map=(0,)),
                    slice_sizes=(1,),
                    mode=lax.GatherScatterMode.PROMISE_IN_BOUNDS)
                rows_buf[e, :] = rows_buf[e, :] * scale
            return 0
        lax.fori_loop(0, CHUNK // 16, scale_body, 0)

        @pl.when(g > 0)
        def _():
            drain_scatter()

        @pl.when(g < NC - 1)
        def _():
            drain_idx()
            issue_gathers(g + 1)

        issue_scatters(g)

        @pl.when(g < NC - 2)
        def _():
            stage_idx(g + 2)
        return 0

    lax.fori_loop(0, NC, chunk_body, 0)
    drain_scatter()
    plsc.subcore_barrier()

    _row_range_copy(
        lambda lo: pltpu.sync_copy(
            acc.at[pl.ds(lo, ROWS_BIG)],
            out.at[pl.ds(pl.multiple_of(c * N + lo, 8), ROWS_BIG)]),
        lambda lo: pltpu.sync_copy(
            acc.at[pl.ds(lo, ROWS_SMALL)],
            out.at[pl.ds(pl.multiple_of(c * N + lo, 8), ROWS_SMALL)]))


@jax.jit
def _propagate_layer(table, colsb, rows3, valsf, zeros):
    mesh = plsc.VectorSubcoreMesh(core_axis_name="c", subcore_axis_name="s")
    return pl.kernel(
        _layer_body,
        out_type=jax.ShapeDtypeStruct((2 * N, H), jnp.float32),
        mesh=mesh,
        scratch_types=[
            pltpu.VMEM((3 * BLK_PER_CHUNK, 128), jnp.int32),
            pltpu.VMEM((3 * BLK_PER_CHUNK, 128), jnp.int32),
            pltpu.VMEM((3 * CHUNK,), jnp.float32),
            pltpu.VMEM((2 * CHUNK, H), jnp.float32),
            pltpu.VMEM_SHARED((N, H), jnp.float32),
            pltpu.SemaphoreType.DMA,
            pltpu.SemaphoreType.DMA,
            pltpu.SemaphoreType.DMA,
        ],
        compiler_params=pltpu.CompilerParams(use_tc_tiling_on_sc=False),
    )(table, colsb, rows3, valsf, zeros)


def _prep_edges(edge_index, edge_vals):
    pad = EP - E
    rows = jnp.concatenate([edge_index[0], jnp.zeros((pad,), jnp.int32)])
    cols = jnp.concatenate([edge_index[1], jnp.zeros((pad,), jnp.int32)])
    vals = jnp.concatenate([edge_vals, jnp.zeros((pad,), jnp.float32)])
    colsb = jnp.concatenate([cols, cols + N]).reshape(2 * NBLK, 128)
    rows3 = rows.reshape(NBLK, 128)
    return colsb, rows3, vals


def _split(x):
    return x.reshape(N, 2, H).transpose(1, 0, 2).reshape(2 * N, H)


def _unsplit(y):
    return y.reshape(2, N, H).transpose(1, 0, 2).reshape(N, D)


def kernel(user_emb, item_emb, edge_index1, edge_vals1,
           edge_index2, edge_vals2, edge_index3, edge_vals3):
    zeros = jnp.zeros((ROWS_BIG, H), jnp.float32)
    ego = _split(jnp.concatenate([user_emb, item_emb], axis=0))

    stage_outs = []
    for ei, ev in ((edge_index3, edge_vals3),
                   (edge_index2, edge_vals2),
                   (edge_index1, edge_vals1)):
        colsb, rows3, vals = _prep_edges(ei, ev)
        a1 = _propagate_layer(ego, colsb, rows3, vals, zeros)
        a2 = _propagate_layer(a1, colsb, rows3, vals, zeros)
        ego = (ego + a1 + a2) * (1.0 / 3.0)
        stage_outs.append(ego)

    total = _unsplit(stage_outs[0] + stage_outs[1] + stage_outs[2])
    return total[:N_USERS], total[N_USERS:]


# speedup vs baseline: 10.6850x; 1.0010x over previous
import functools

import jax
import jax.numpy as jnp
from jax import lax
from jax.experimental import pallas as pl
from jax.experimental.pallas import tpu as pltpu
from jax.experimental.pallas import tpu_sc as plsc

N_USERS = 50000
N = 100000
D = 32
H = 16
E = 1600000
EP = 1638400
NBLK = EP // 128
TILES = 16
EDGES_PER_TILE = EP // TILES
CHUNK = 512
BLK_PER_CHUNK = CHUNK // 128
CHUNKS_PER_TILE = EDGES_PER_TILE // CHUNK
GATHER_BYTES = CHUNK * H * 4
SCATTER_BYTES = CHUNK * H * 4
IDX_BYTES = 3 * CHUNK * 4
ROWS_BIG = 6256
ROWS_SMALL = 6248


def _layer_body(table, colsb, rows3, valsf, zeros, out,
                gidx_v, sidx_v, vals_v, rows_buf, acc, gsem, ssem, isem):
    c = lax.axis_index("c")
    s = lax.axis_index("s")

    def _row_range_copy(fn_big, fn_small):
        @pl.when(s < 4)
        def _():
            lo = pl.multiple_of(s * ROWS_BIG, 8)
            fn_big(lo)

        @pl.when(s >= 4)
        def _():
            lo = pl.multiple_of(4 * ROWS_BIG + (s - 4) * ROWS_SMALL, 8)
            fn_small(lo)

    _row_range_copy(
        lambda lo: pltpu.sync_copy(zeros.at[pl.ds(0, ROWS_BIG)],
                                   acc.at[pl.ds(lo, ROWS_BIG)]),
        lambda lo: pltpu.sync_copy(zeros.at[pl.ds(0, ROWS_SMALL)],
                                   acc.at[pl.ds(lo, ROWS_SMALL)]))
    plsc.subcore_barrier()

    NC = CHUNKS_PER_TILE

    def drain_gather():
        pltpu.make_async_copy(table.at[pl.ds(0, CHUNK)],
                              rows_buf.at[pl.ds(0, CHUNK)], gsem).wait()

    def drain_scatter():
        pltpu.make_async_copy(table.at[pl.ds(0, CHUNK)],
                              rows_buf.at[pl.ds(0, CHUNK)], ssem).wait()

    def drain_idx():
        pltpu.make_async_copy(valsf.at[pl.ds(0, 3 * CHUNK)], vals_v,
                              isem).wait()

    def stage_idx(g):
        st = lax.rem(g, 3)
        ebase = s * EDGES_PER_TILE + g * CHUNK
        pltpu.async_copy(colsb.at[pl.ds(c * EP + ebase, CHUNK)],
                         gidx_v.at[st], isem)
        pltpu.async_copy(rows3.at[pl.ds(ebase, CHUNK)],
                         sidx_v.at[st], isem)
        pltpu.async_copy(valsf.at[pl.ds(ebase, CHUNK)],
                         vals_v.at[pl.ds(st * CHUNK, CHUNK)], isem)

    def issue_gathers(g):
        st = lax.rem(g, 3)
        b = lax.rem(g, 2)
        pltpu.async_copy(table.at[gidx_v.at[st]],
                         rows_buf.at[pl.ds(b * CHUNK, CHUNK)], gsem)

    def issue_scatters(g):
        st = lax.rem(g, 3)
        b = lax.rem(g, 2)
        pltpu.async_copy(rows_buf.at[pl.ds(b * CHUNK, CHUNK)],
                         acc.at[sidx_v.at[st]], ssem, add=True)

    stage_idx(jnp.int32(0))
    drain_idx()
    issue_gathers(jnp.int32(0))
    stage_idx(jnp.int32(1))

    def chunk_body(g, _):
        st = lax.rem(g, 3)
        b = lax.rem(g, 2)
        drain_gather()

        def scale_body(k, _):
            v16 = vals_v[pl.ds(st * CHUNK + k * 16, 16)]
            base = b * CHUNK + k * 16
            for i in range(16):
                e = base + i
                scale = lax.gather(
                    v16, jnp.full((16, 1), i, jnp.int32),
                    lax.GatherDimensionNumbers(offset_dims=(),
                                               collapsed_slice_dims=(0,),
                                               start_index_map=(0,)),
                    slice_sizes=(1,),
                    mode=lax.GatherScatterMode.PROMISE_IN_BOUNDS)
                rows_buf[e, :] = rows_buf[e, :] * scale
            return 0
        lax.fori_loop(0, CHUNK // 16, scale_body, 0)

        @pl.when(g > 0)
        def _():
            drain_scatter()

        @pl.when(g < NC - 1)
        def _():
            drain_idx()
            issue_gathers(g + 1)

        issue_scatters(g)

        @pl.when(g < NC - 2)
        def _():
            stage_idx(g + 2)
        return 0

    lax.fori_loop(0, NC, chunk_body, 0)
    drain_scatter()
    plsc.subcore_barrier()

    _row_range_copy(
        lambda lo: pltpu.sync_copy(
            acc.at[pl.ds(lo, ROWS_BIG)],
            out.at[pl.ds(pl.multiple_of(c * N + lo, 8), ROWS_BIG)]),
        lambda lo: pltpu.sync_copy(
            acc.at[pl.ds(lo, ROWS_SMALL)],
            out.at[pl.ds(pl.multiple_of(c * N + lo, 8), ROWS_SMALL)]))


@jax.jit
def _propagate_layer(table, colsb, rows3, valsf, zeros):
    mesh = plsc.VectorSubcoreMesh(core_axis_name="c", subcore_axis_name="s")
    return pl.kernel(
        _layer_body,
        out_type=jax.ShapeDtypeStruct((2 * N, H), jnp.float32),
        mesh=mesh,
        scratch_types=[
            pltpu.VMEM((3, CHUNK), jnp.int32),
            pltpu.VMEM((3, CHUNK), jnp.int32),
            pltpu.VMEM((3 * CHUNK,), jnp.float32),
            pltpu.VMEM((2 * CHUNK, H), jnp.float32),
            pltpu.VMEM_SHARED((N, H), jnp.float32),
            pltpu.SemaphoreType.DMA,
            pltpu.SemaphoreType.DMA,
            pltpu.SemaphoreType.DMA,
        ],
        compiler_params=pltpu.CompilerParams(use_tc_tiling_on_sc=False),
    )(table, colsb, rows3, valsf, zeros)


def _prep_edges(edge_index, edge_vals):
    pad = EP - E
    rows = jnp.concatenate([edge_index[0], jnp.zeros((pad,), jnp.int32)])
    cols = jnp.concatenate([edge_index[1], jnp.zeros((pad,), jnp.int32)])
    vals = jnp.concatenate([edge_vals, jnp.zeros((pad,), jnp.float32)])
    colsb = jnp.concatenate([cols, cols + N])
    return colsb, rows, vals


def _split(x):
    return x.reshape(N, 2, H).transpose(1, 0, 2).reshape(2 * N, H)


def _unsplit(y):
    return y.reshape(2, N, H).transpose(1, 0, 2).reshape(N, D)


def kernel(user_emb, item_emb, edge_index1, edge_vals1,
           edge_index2, edge_vals2, edge_index3, edge_vals3):
    zeros = jnp.zeros((ROWS_BIG, H), jnp.float32)
    ego = _split(jnp.concatenate([user_emb, item_emb], axis=0))

    stage_outs = []
    for ei, ev in ((edge_index3, edge_vals3),
                   (edge_index2, edge_vals2),
                   (edge_index1, edge_vals1)):
        colsb, rows3, vals = _prep_edges(ei, ev)
        a1 = _propagate_layer(ego, colsb, rows3, vals, zeros)
        a2 = _propagate_layer(a1, colsb, rows3, vals, zeros)
        ego = (ego + a1 + a2) * (1.0 / 3.0)
        stage_outs.append(ego)

    total = _unsplit(stage_outs[0] + stage_outs[1] + stage_outs[2])
    return total[:N_USERS], total[N_USERS:]


# speedup vs baseline: 12.8051x; 1.1984x over previous
import functools

import jax
import jax.numpy as jnp
from jax import lax
from jax.experimental import pallas as pl
from jax.experimental.pallas import tpu as pltpu
from jax.experimental.pallas import tpu_sc as plsc

N_USERS = 50000
N = 100000
D = 32
H = 16
E = 1600000
EP = 1638400
NBLK = EP // 128
TILES = 16
EDGES_PER_TILE = EP // TILES
CHUNK = 256
RBUF = 4
ISET = 6
CHUNKS_PER_TILE = EDGES_PER_TILE // CHUNK
GATHER_BYTES = CHUNK * H * 4
SCATTER_BYTES = CHUNK * H * 4
IDX_BYTES = 3 * CHUNK * 4
ROWS_BIG = 6256
ROWS_SMALL = 6248


def _layer_body(table, colsb, rows3, valsf, zeros, out,
                gidx_v, sidx_v, vals_v, rows_buf, acc, gsem, ssem, isem):
    c = lax.axis_index("c")
    s = lax.axis_index("s")

    def _row_range_copy(fn_big, fn_small):
        @pl.when(s < 4)
        def _():
            lo = pl.multiple_of(s * ROWS_BIG, 8)
            fn_big(lo)

        @pl.when(s >= 4)
        def _():
            lo = pl.multiple_of(4 * ROWS_BIG + (s - 4) * ROWS_SMALL, 8)
            fn_small(lo)

    _row_range_copy(
        lambda lo: pltpu.sync_copy(zeros.at[pl.ds(0, ROWS_BIG)],
                                   acc.at[pl.ds(lo, ROWS_BIG)]),
        lambda lo: pltpu.sync_copy(zeros.at[pl.ds(0, ROWS_SMALL)],
                                   acc.at[pl.ds(lo, ROWS_SMALL)]))
    plsc.subcore_barrier()

    NC = CHUNKS_PER_TILE

    def drain_gather():
        pltpu.make_async_copy(table.at[pl.ds(0, CHUNK)],
                              rows_buf.at[pl.ds(0, CHUNK)], gsem).wait()

    def drain_scatter():
        pltpu.make_async_copy(table.at[pl.ds(0, CHUNK)],
                              rows_buf.at[pl.ds(0, CHUNK)], ssem).wait()

    def drain_idx():
        pltpu.make_async_copy(valsf.at[pl.ds(0, 3 * CHUNK)],
                              vals_v.at[pl.ds(0, 3 * CHUNK)], isem).wait()

    def stage_idx(g):
        st = lax.rem(g, ISET)
        ebase = s * EDGES_PER_TILE + g * CHUNK
        pltpu.async_copy(colsb.at[pl.ds(c * EP + ebase, CHUNK)],
                         gidx_v.at[st], isem)
        pltpu.async_copy(rows3.at[pl.ds(ebase, CHUNK)],
                         sidx_v.at[st], isem)
        pltpu.async_copy(valsf.at[pl.ds(ebase, CHUNK)],
                         vals_v.at[pl.ds(st * CHUNK, CHUNK)], isem)

    def issue_gathers(g):
        st = lax.rem(g, ISET)
        b = lax.rem(g, RBUF)
        pltpu.async_copy(table.at[gidx_v.at[st]],
                         rows_buf.at[pl.ds(b * CHUNK, CHUNK)], gsem)

    def issue_scatters(g):
        st = lax.rem(g, ISET)
        b = lax.rem(g, RBUF)
        pltpu.async_copy(rows_buf.at[pl.ds(b * CHUNK, CHUNK)],
                         acc.at[sidx_v.at[st]], ssem, add=True)

    for h in range(ISET - 1):
        stage_idx(jnp.int32(h))
    for h in range(RBUF - 1):
        drain_idx()
        issue_gathers(jnp.int32(h))

    def chunk_body(g, _):
        st = lax.rem(g, ISET)
        b = lax.rem(g, RBUF)
        drain_gather()

        def scale_body(k, _):
            v16 = vals_v[pl.ds(st * CHUNK + k * 16, 16)]
            base = b * CHUNK + k * 16
            for i in range(16):
                e = base + i
                scale = lax.gather(
                    v16, jnp.full((16, 1), i, jnp.int32),
                    lax.GatherDimensionNumbers(offset_dims=(),
                                               collapsed_slice_dims=(0,),
                                               start_index_map=(0,)),
                    slice_sizes=(1,),
                    mode=lax.GatherScatterMode.PROMISE_IN_BOUNDS)
                rows_buf[e, :] = rows_buf[e, :] * scale
            return 0
        lax.fori_loop(0, CHUNK // 16, scale_body, 0)

        @pl.when(g > 0)
        def _():
            drain_scatter()

        @pl.when(g < NC - RBUF + 1)
        def _():
            drain_idx()
            issue_gathers(g + RBUF - 1)

        issue_scatters(g)

        @pl.when(g < NC - ISET + 1)
        def _():
            stage_idx(g + ISET - 1)
        return 0

    lax.fori_loop(0, NC, chunk_body, 0)
    drain_scatter()
    plsc.subcore_barrier()

    _row_range_copy(
        lambda lo: pltpu.sync_copy(
            acc.at[pl.ds(lo, ROWS_BIG)],
            out.at[pl.ds(pl.multiple_of(c * N + lo, 8), ROWS_BIG)]),
        lambda lo: pltpu.sync_copy(
            acc.at[pl.ds(lo, ROWS_SMALL)],
            out.at[pl.ds(pl.multiple_of(c * N + lo, 8), ROWS_SMALL)]))


@jax.jit
def _propagate_layer(table, colsb, rows3, valsf, zeros):
    mesh = plsc.VectorSubcoreMesh(core_axis_name="c", subcore_axis_name="s")
    return pl.kernel(
        _layer_body,
        out_type=jax.ShapeDtypeStruct((2 * N, H), jnp.float32),
        mesh=mesh,
        scratch_types=[
            pltpu.VMEM((ISET, CHUNK), jnp.int32),
            pltpu.VMEM((ISET, CHUNK), jnp.int32),
            pltpu.VMEM((ISET * CHUNK,), jnp.float32),
            pltpu.VMEM((RBUF * CHUNK, H), jnp.float32),
            pltpu.VMEM_SHARED((N, H), jnp.float32),
            pltpu.SemaphoreType.DMA,
            pltpu.SemaphoreType.DMA,
            pltpu.SemaphoreType.DMA,
        ],
        compiler_params=pltpu.CompilerParams(use_tc_tiling_on_sc=False),
    )(table, colsb, rows3, valsf, zeros)


def _prep_edges(edge_index, edge_vals):
    pad = EP - E
    rows = jnp.concatenate([edge_index[0], jnp.zeros((pad,), jnp.int32)])
    cols = jnp.concatenate([edge_index[1], jnp.zeros((pad,), jnp.int32)])
    vals = jnp.concatenate([edge_vals, jnp.zeros((pad,), jnp.float32)])
    colsb = jnp.concatenate([cols, cols + N])
    return colsb, rows, vals


def _split(x):
    return x.reshape(N, 2, H).transpose(1, 0, 2).reshape(2 * N, H)


def _unsplit(y):
    return y.reshape(2, N, H).transpose(1, 0, 2).reshape(N, D)


def kernel(user_emb, item_emb, edge_index1, edge_vals1,
           edge_index2, edge_vals2, edge_index3, edge_vals3):
    zeros = jnp.zeros((ROWS_BIG, H), jnp.float32)
    ego = _split(jnp.concatenate([user_emb, item_emb], axis=0))

    stage_outs = []
    for ei, ev in ((edge_index3, edge_vals3),
                   (edge_index2, edge_vals2),
                   (edge_index1, edge_vals1)):
        colsb, rows3, vals = _prep_edges(ei, ev)
        a1 = _propagate_layer(ego, colsb, rows3, vals, zeros)
        a2 = _propagate_layer(a1, colsb, rows3, vals, zeros)
        ego = (ego + a1 + a2) * (1.0 / 3.0)
        stage_outs.append(ego)

    total = _unsplit(stage_outs[0] + stage_outs[1] + stage_outs[2])
    return total[:N_USERS], total[N_USERS:]
